# baseline (speedup 1.0000x reference)
import functools

import jax
import jax.numpy as jnp
from jax import lax
from jax.experimental import pallas as pl
from jax.experimental.pallas import tpu as pltpu
from jax.experimental.pallas import tpu_sc as plsc

_NC, _NS, _L = 2, 16, 16
_NW = _NC * _NS

def _pack_body(a_ref, b_ref, xy_ref, z_ref):
    d = a_ref[...] - b_ref[...]
    dx, dy, dz = d[0:1], d[1:2], d[2:3]
    xu = lax.bitcast_convert_type(dx.astype(jnp.bfloat16), jnp.uint16)
    yu = lax.bitcast_convert_type(dy.astype(jnp.bfloat16), jnp.uint16)
    xy = yu.astype(jnp.uint32) << 16 | xu.astype(jnp.uint32)
    xy_ref[...] = lax.bitcast_convert_type(xy, jnp.int32)
    z_ref[...] = dz.astype(jnp.bfloat16)


def _lo(g):
    return plsc.bitcast(jnp.left_shift(g, 16), jnp.float32)


def _hi(g):
    return plsc.bitcast(jnp.bitwise_and(g, -65536), jnp.float32)


def _sc_body(npad, chunk, k_nn, xy_hbm, zp_hbm, idx_hbm, lg_hbm, out_hbm,
             table_v, idx_v, w_v, acc_v, sem):
    ngroups = chunk // _L
    wid = lax.axis_index("s") * _NC + lax.axis_index("c")
    base = wid * chunk
    kc = chunk * k_nn

    cp = pltpu.async_copy(xy_hbm, table_v, sem)
    pltpu.sync_copy(idx_hbm.at[pl.ds(wid * kc, kc)], idx_v)
    pltpu.sync_copy(lg_hbm.at[pl.ds(wid * kc, kc)], w_v)

    def smax_body(j, carry):
        off = j * _L
        ls = [w_v[pl.ds(k * chunk + off, _L)] for k in range(k_nn)]
        m = ls[0]
        for l in ls[1:]:
            m = jnp.maximum(m, l)
        es = [jnp.exp(l - m) for l in ls]
        s = es[0]
        for e in es[1:]:
            s = s + e
        inv = 1.0 / s
        for k in range(k_nn):
            w_v[pl.ds(k * chunk + off, _L)] = es[k] * inv
        return carry
    lax.fori_loop(0, ngroups, smax_body, 0)

    cp.wait()

    def xy_body(j, acc):
        off = j * _L
        own = table_v[pl.ds(base + off, _L)]
        rx, ry = _lo(own), _hi(own)
        for k in range(k_nn):
            ik = idx_v[pl.ds(k * chunk + off, _L)]
            wk = w_v[pl.ds(k * chunk + off, _L)]
            g = plsc.load_gather(table_v, [ik])
            rx = rx - wk * _lo(g)
            ry = ry - wk * _hi(g)
        return acc + rx * rx + ry * ry
    total = lax.fori_loop(0, ngroups, xy_body, jnp.zeros((_L,), jnp.float32))

    pltpu.sync_copy(zp_hbm, table_v.at[pl.ds(0, npad // 2)])
    iota2 = lax.iota(jnp.int32, _L) * 2

    def z_body(j, acc):
        off2 = j * 2 * _L
        zw = table_v[pl.ds((base + off2) // 2, _L)]
        r_ev, r_od = _lo(zw), _hi(zw)
        for k in range(k_nn):
            kb = k * chunk + off2
            i_ev = plsc.load_gather(idx_v, [iota2 + kb])
            i_od = plsc.load_gather(idx_v, [iota2 + (kb + 1)])
            w_ev = plsc.load_gather(w_v, [iota2 + kb])
            w_od = plsc.load_gather(w_v, [iota2 + (kb + 1)])
            g_ev = plsc.load_gather(table_v, [jnp.right_shift(i_ev, 1)])
            g_od = plsc.load_gather(table_v, [jnp.right_shift(i_od, 1)])
            z_ev = jnp.where(jnp.bitwise_and(i_ev, 1) == 1, _hi(g_ev), _lo(g_ev))
            z_od = jnp.where(jnp.bitwise_and(i_od, 1) == 1, _hi(g_od), _lo(g_od))
            r_ev = r_ev - w_ev * z_ev
            r_od = r_od - w_od * z_od
        return acc + r_ev * r_ev + r_od * r_od
    total = lax.fori_loop(0, ngroups // 2, z_body, total)

    acc_v[...] = total
    pltpu.sync_copy(acc_v, out_hbm.at[pl.ds(wid * _L, _L)])


def kernel(pcl, prev_pcl, weight_logits, nn_idxs):
    n = pcl.shape[0]
    k_nn = nn_idxs.shape[1]
    npad = -(-n // (_NW * _L * 2)) * (_NW * _L * 2)
    chunk = npad // _NW
    pad = npad - n

    pclT = jnp.pad(pcl, ((0, pad), (0, 0))).T
    prevT = jnp.pad(prev_pcl, ((0, pad), (0, 0))).T
    xy, zb = pl.pallas_call(
        _pack_body,
        out_shape=(jax.ShapeDtypeStruct((1, npad), jnp.int32),
                   jax.ShapeDtypeStruct((1, npad), jnp.bfloat16)),
    )(pclT, prevT)
    zp = lax.bitcast_convert_type(zb.reshape(npad // 2, 2), jnp.int32)

    idxB = jnp.pad(nn_idxs.astype(jnp.int32), ((0, pad), (0, 0)),
                   constant_values=n)
    idxB = idxB.reshape(_NW, chunk, k_nn).transpose(0, 2, 1).reshape(-1)
    lgB = jnp.pad(weight_logits, ((0, pad), (0, 0)))
    lgB = lgB.reshape(_NW, chunk, k_nn).transpose(0, 2, 1).reshape(-1)

    mesh = plsc.VectorSubcoreMesh(core_axis_name="c", subcore_axis_name="s")
    partials = pl.kernel(
        functools.partial(_sc_body, npad, chunk, k_nn),
        out_type=jax.ShapeDtypeStruct((_NW * _L,), jnp.float32),
        mesh=mesh,
        compiler_params=pltpu.CompilerParams(
            needs_layout_passes=False, use_tc_tiling_on_sc=False),
        scratch_types=[
            pltpu.VMEM((npad,), jnp.int32),
            pltpu.VMEM((k_nn * chunk,), jnp.int32),
            pltpu.VMEM((k_nn * chunk,), jnp.float32),
            pltpu.VMEM((_L,), jnp.float32),
            pltpu.SemaphoreType.DMA,
        ],
    )(xy.reshape(-1), zp, idxB, lgB)

    return jnp.sum(partials) / (3.0 * n)
